# Initial kernel scaffold; baseline (speedup 1.0000x reference)
import jax
import jax.numpy as jnp
from jax.experimental import pallas as pl


def kernel(x, edge_index, edge_label_index, W1, b1, W2, b2):
    raise NotImplementedError("write your pallas kernel here")


# speedup vs baseline: 8.0960x; 8.0960x over previous
import functools

import jax
import jax.numpy as jnp
from jax import lax
from jax.experimental import pallas as pl
from jax.experimental.pallas import tpu as pltpu
from jax.experimental.pallas import tpu_sc as plsc

NC = 2
NS = 16
D = 128
BLK = 256
CW = 128


def _mesh():
    return plsc.VectorSubcoreMesh(core_axis_name="c", subcore_axis_name="s")


def _sc_cnt(n_pad, ch):
    zrows = n_pad // NS

    def body(dst_hbm, zeros_hbm, ones_hbm, cnt_out, didx_v, ones_v, cnt_sh, sem):
        c = lax.axis_index("c")
        s = lax.axis_index("s")
        pltpu.sync_copy(dst_hbm.at[c, s], didx_v)
        pltpu.sync_copy(ones_hbm, ones_v)
        pltpu.sync_copy(zeros_hbm, cnt_sh.at[pl.ds(s * zrows, zrows)])
        plsc.subcore_barrier()

        def step(j, carry):
            pltpu.sync_copy(ones_v, cnt_sh.at[didx_v.at[j]], add=True)
            return carry

        lax.fori_loop(0, ch, step, 0)
        plsc.subcore_barrier()
        pltpu.sync_copy(cnt_sh.at[pl.ds(s * zrows, zrows)],
                        cnt_out.at[c, pl.ds(s * zrows, zrows)])

    return pl.kernel(
        body,
        out_type=jax.ShapeDtypeStruct((NC, n_pad, D), jnp.float32),
        mesh=_mesh(),
        compiler_params=pltpu.CompilerParams(needs_layout_passes=False),
        scratch_types=[
            pltpu.VMEM((ch, CW), jnp.int32),
            pltpu.VMEM((CW, D), jnp.float32),
            pltpu.VMEM_SHARED((n_pad, D), jnp.float32),
            pltpu.SemaphoreType.DMA,
        ],
    )


def _sc_scatter(n_pad, ch):
    zrows = n_pad // NS

    def body(u_hbm, src_hbm, dst_hbm, zeros_hbm, acc_out,
             sidx_v, didx_v, rows_v, acc_sh, sem):
        c = lax.axis_index("c")
        s = lax.axis_index("s")
        pltpu.sync_copy(src_hbm.at[c, s], sidx_v)
        pltpu.sync_copy(dst_hbm.at[c, s], didx_v)
        pltpu.sync_copy(zeros_hbm, acc_sh.at[pl.ds(s * zrows, zrows)])
        plsc.subcore_barrier()

        def step(j, carry):
            pltpu.async_copy(u_hbm.at[sidx_v.at[j]], rows_v, sem).wait()
            pltpu.sync_copy(rows_v, acc_sh.at[didx_v.at[j]], add=True)
            return carry

        lax.fori_loop(0, ch, step, 0)
        plsc.subcore_barrier()
        pltpu.sync_copy(acc_sh.at[pl.ds(s * zrows, zrows)],
                        acc_out.at[c, pl.ds(s * zrows, zrows)])

    return pl.kernel(
        body,
        out_type=jax.ShapeDtypeStruct((NC, n_pad, D), jnp.float32),
        mesh=_mesh(),
        compiler_params=pltpu.CompilerParams(needs_layout_passes=False),
        scratch_types=[
            pltpu.VMEM((ch, CW), jnp.int32),
            pltpu.VMEM((ch, CW), jnp.int32),
            pltpu.VMEM((CW, D), jnp.float32),
            pltpu.VMEM_SHARED((n_pad, D), jnp.float32),
            pltpu.SemaphoreType.DMA,
        ],
    )


def _sc_decode(ch2):
    def body(z_hbm, sidx_hbm, ridx_hbm, out_hbm,
             sidx_v, ridx_v, srows_v, rrows_v, out_v, sem):
        c = lax.axis_index("c")
        s = lax.axis_index("s")
        pltpu.sync_copy(sidx_hbm.at[c, s], sidx_v)
        pltpu.sync_copy(ridx_hbm.at[c, s], ridx_v)

        lanes = lax.iota(jnp.int32, 16)

        def step(j, carry):
            pltpu.async_copy(z_hbm.at[sidx_v.at[j]], srows_v, sem).wait()
            pltpu.async_copy(z_hbm.at[ridx_v.at[j]], rrows_v, sem).wait()

            def gdot(g, carry2):
                def edot(i, acc16):
                    e = g * 16 + i
                    p = srows_v[e, pl.ds(0, 16)] * rrows_v[e, pl.ds(0, 16)]
                    for k in range(1, 8):
                        p = p + (srows_v[e, pl.ds(k * 16, 16)] *
                                 rrows_v[e, pl.ds(k * 16, 16)])
                    return jnp.where(lanes == i, jnp.sum(p), acc16)

                acc16 = lax.fori_loop(0, 16, edot,
                                      jnp.zeros((16,), jnp.float32))
                out_v[j, pl.ds(g * 16, 16)] = acc16
                return carry2

            lax.fori_loop(0, CW // 16, gdot, 0)
            return carry

        lax.fori_loop(0, ch2, step, 0)
        pltpu.sync_copy(out_v, out_hbm.at[c, s])

    return pl.kernel(
        body,
        out_type=jax.ShapeDtypeStruct((NC, NS, ch2, CW), jnp.float32),
        mesh=_mesh(),
        compiler_params=pltpu.CompilerParams(needs_layout_passes=False),
        scratch_types=[
            pltpu.VMEM((ch2, CW), jnp.int32),
            pltpu.VMEM((ch2, CW), jnp.int32),
            pltpu.VMEM((CW, D), jnp.float32),
            pltpu.VMEM((CW, D), jnp.float32),
            pltpu.VMEM((ch2, CW), jnp.float32),
            pltpu.SemaphoreType.DMA,
        ],
    )


def _dis(c0, c1):
    return lax.rsqrt(1.0 + c0 + c1)


def _tc_mm1_body(x_ref, w_ref, c0_ref, c1_ref, o_ref):
    dis = _dis(c0_ref[...], c1_ref[...])
    o_ref[...] = dis * jnp.dot(x_ref[...], w_ref[...],
                               preferred_element_type=jnp.float32)


def _tc_mm2_body(a0_ref, a1_ref, u1_ref, c0_ref, c1_ref, b1_ref, w2_ref, o_ref):
    dis = _dis(c0_ref[...], c1_ref[...])
    h = dis * (a0_ref[...] + a1_ref[...] + u1_ref[...]) + b1_ref[...]
    h = jnp.maximum(h, 0.0)
    o_ref[...] = dis * jnp.dot(h, w2_ref[...],
                               preferred_element_type=jnp.float32)


def _tc_fin_body(a0_ref, a1_ref, u2_ref, c0_ref, c1_ref, b2_ref, o_ref):
    dis = _dis(c0_ref[...], c1_ref[...])
    o_ref[...] = dis * (a0_ref[...] + a1_ref[...] + u2_ref[...]) + b2_ref[...]


def _row_spec():
    return pl.BlockSpec((BLK, D), lambda i: (i, 0))


def _full_spec():
    return pl.BlockSpec((D, D), lambda i: (0, 0))


def _bias_spec():
    return pl.BlockSpec((1, D), lambda i: (0, 0))


def _tc_call(body, n_pad, in_specs):
    return pl.pallas_call(
        body,
        grid=(n_pad // BLK,),
        in_specs=in_specs,
        out_specs=_row_spec(),
        out_shape=jax.ShapeDtypeStruct((n_pad, D), jnp.float32),
    )


def kernel(x, edge_index, edge_label_index, W1, b1, W2, b2):
    n, d = x.shape
    e = edge_index.shape[1]
    el = edge_label_index.shape[1]
    assert d == D

    n_pad = ((n + BLK - 1) // BLK) * BLK
    dummy = n

    epc = NC * NS * CW
    ch = (e + epc - 1) // epc
    e_pad = ch * epc
    ch2 = (el + epc - 1) // epc
    el_pad = ch2 * epc

    x_p = jnp.pad(x, ((0, n_pad - n), (0, 0)))
    src_r = jnp.pad(edge_index[0], (0, e_pad - e)).reshape(NC, NS, ch, CW)
    dst_r = jnp.pad(edge_index[1], (0, e_pad - e),
                    constant_values=dummy).reshape(NC, NS, ch, CW)
    sidx_r = jnp.pad(edge_label_index[0], (0, el_pad - el)).reshape(NC, NS, ch2, CW)
    ridx_r = jnp.pad(edge_label_index[1], (0, el_pad - el)).reshape(NC, NS, ch2, CW)

    zeros_rows = jnp.zeros((n_pad // NS, D), jnp.float32)
    ones_rows = jnp.ones((CW, D), jnp.float32)
    b1r = b1.reshape(1, D)
    b2r = b2.reshape(1, D)

    cnt = _sc_cnt(n_pad, ch)(dst_r, zeros_rows, ones_rows)
    c0, c1 = cnt[0], cnt[1]

    u1 = _tc_call(_tc_mm1_body, n_pad,
                  [_row_spec(), _full_spec(), _row_spec(), _row_spec()])(
        x_p, W1, c0, c1)
    a1 = _sc_scatter(n_pad, ch)(u1, src_r, dst_r, zeros_rows)

    u2 = _tc_call(_tc_mm2_body, n_pad,
                  [_row_spec(), _row_spec(), _row_spec(), _row_spec(),
                   _row_spec(), _bias_spec(), _full_spec()])(
        a1[0], a1[1], u1, c0, c1, b1r, W2)
    a2 = _sc_scatter(n_pad, ch)(u2, src_r, dst_r, zeros_rows)

    z = _tc_call(_tc_fin_body, n_pad,
                 [_row_spec(), _row_spec(), _row_spec(), _row_spec(),
                  _row_spec(), _bias_spec()])(
        a2[0], a2[1], u2, c0, c1, b2r)

    dots = _sc_decode(ch2)(z, sidx_r, ridx_r)
    return dots.reshape(-1)[:el]
